# baseline (device time: 65499 ns/iter reference)
import jax
import jax.numpy as jnp
from jax import lax
from jax.experimental import pallas as pl
from jax.experimental.pallas import tpu as pltpu

N_DEV = 8
B = 2
SQ = 512
SKV = 512
HQ = 64
HQ_LOC = 8
DH = 64
D_MODEL = 768
HBLK = HQ_LOC * DH


def kernel(x, Wq, K_ext, V_ext, Wo):
    k_flat = K_ext.reshape(B, SKV, HQ * DH)
    v_flat = V_ext.reshape(B, SKV, HQ * DH)

    def body(x_ref, wq_ref, k_any_ref, v_any_ref, wo_ref, out_ref,
             acc_ref, rx_ref, ctx_ref, k_ref, v_ref,
             kv_sems, send_sems, recv_sems):
        my = lax.axis_index("i")
        CT = D_MODEL // 3

        kdma = pltpu.make_async_copy(
            k_any_ref.at[:, :, pl.ds(my * HBLK, HBLK)], k_ref, kv_sems.at[0])
        vdma = pltpu.make_async_copy(
            v_any_ref.at[:, :, pl.ds(my * HBLK, HBLK)], v_ref, kv_sems.at[1])
        kdma.start()
        vdma.start()

        barrier_sem = pltpu.get_barrier_semaphore()
        for mask in (1, 3, 4):
            pl.semaphore_signal(
                barrier_sem, inc=1,
                device_id=(jnp.bitwise_xor(my, mask),),
                device_id_type=pl.DeviceIdType.MESH)
        pl.semaphore_wait(barrier_sem, 3)

        x2 = x_ref[:].reshape(B * SQ, D_MODEL)
        q = jnp.dot(x2, wq_ref[:], preferred_element_type=jnp.float32)
        kdma.wait()
        vdma.wait()

        qi = lax.broadcasted_iota(jnp.int32, (SQ, SKV), 0)
        ki = lax.broadcasted_iota(jnp.int32, (SQ, SKV), 1)
        maskf = ((jnp.abs(qi - ki) <= 128) | (ki < 32) | (qi < 32)
                 ).astype(jnp.float32)

        for b in range(B):
            for h in range(HQ_LOC):
                q_bh = q[b * SQ:(b + 1) * SQ, h * DH:(h + 1) * DH]
                k_bh = k_ref[b, :, h * DH:(h + 1) * DH]
                v_bh = v_ref[b, :, h * DH:(h + 1) * DH]
                s = lax.dot_general(
                    q_bh, k_bh, (((1,), (1,)), ((), ())),
                    preferred_element_type=jnp.float32) * 0.125
                e = jnp.exp(s) * maskf
                ctx_ref[:, h * DH:(h + 1) * DH] = (
                    jnp.dot(e, v_bh, preferred_element_type=jnp.float32)
                    / jnp.sum(e, axis=1, keepdims=True))
            acc = jnp.dot(ctx_ref[:], wo_ref[:],
                          preferred_element_type=jnp.float32)
            for t in range(3):
                acc_ref[t, b * SQ:(b + 1) * SQ, :] = (
                    acc[:, CT * t:CT * (t + 1)].astype(jnp.bfloat16))

        d0 = jnp.bitwise_and(my, 1)
        d1 = jnp.bitwise_and(lax.shift_right_logical(my, 1), 1)
        d2 = jnp.bitwise_and(lax.shift_right_logical(my, 2), 1)
        coef = {1: jnp.bitwise_xor(d0, d1), 3: d1, 4: d2}
        perms = ((1, 3, 4), (3, 4, 1), (4, 1, 3))

        stage_params = []
        for t in range(3):
            m0, mA, mB = perms[t]
            al = coef[m0]
            keep = al * 512
            stage_params.append([
                (m0, (1 - al) * 512, 0),
                (mA, keep, 512),
                (mB, keep, 1024),
                (m0, keep, None),
            ])

        def make_rdma(t, s):
            mask, src_off, rx_off = stage_params[t][s]
            src = acc_ref.at[t, pl.ds(src_off, 512)]
            if rx_off is None:
                dst = acc_ref.at[t, pl.ds(src_off, 512)]
            else:
                dst = rx_ref.at[t, pl.ds(rx_off, 512)]
            return pltpu.make_async_remote_copy(
                src_ref=src, dst_ref=dst,
                send_sem=send_sems.at[t * 4 + s],
                recv_sem=recv_sems.at[t * 4 + s],
                device_id=(jnp.bitwise_xor(my, mask),),
                device_id_type=pl.DeviceIdType.MESH,
            )

        rd = [[None] * 4 for _ in range(3)]
        for t in range(3):
            rd[t][0] = make_rdma(t, 0)
            rd[t][0].start()
        for s in range(4):
            for t in range(3):
                rd[t][s].wait()
                if s < 3:
                    keep = stage_params[t][1][1]
                    rx_off = stage_params[t][s][2]
                    acc_ref[t, pl.ds(keep, 512), :] = (
                        acc_ref[t, pl.ds(keep, 512), :]
                        + rx_ref[t, pl.ds(rx_off, 512), :])
                if s < 3:
                    rd[t][s + 1] = make_rdma(t, s + 1)
                    rd[t][s + 1].start()

        for t in range(3):
            out_ref[:, :, CT * t:CT * (t + 1)] = (
                acc_ref[t].reshape(B, SQ, CT).astype(jnp.float32))

    return pl.pallas_call(
        body,
        out_shape=jax.ShapeDtypeStruct((B, SQ, D_MODEL), jnp.float32),
        in_specs=[
            pl.BlockSpec(memory_space=pltpu.VMEM),
            pl.BlockSpec(memory_space=pltpu.VMEM),
            pl.BlockSpec(memory_space=pl.ANY),
            pl.BlockSpec(memory_space=pl.ANY),
            pl.BlockSpec(memory_space=pltpu.VMEM),
        ],
        out_specs=pl.BlockSpec(memory_space=pltpu.VMEM),
        scratch_shapes=[
            pltpu.VMEM((3, B * SQ, D_MODEL // 3), jnp.bfloat16),
            pltpu.VMEM((3, 1536, D_MODEL // 3), jnp.bfloat16),
            pltpu.VMEM((SQ, HBLK), jnp.float32),
            pltpu.VMEM((B, SKV, HBLK), jnp.float32),
            pltpu.VMEM((B, SKV, HBLK), jnp.float32),
            pltpu.SemaphoreType.DMA((2,)),
            pltpu.SemaphoreType.DMA((12,)),
            pltpu.SemaphoreType.DMA((12,)),
        ],
        compiler_params=pltpu.CompilerParams(collective_id=0),
    )(x, Wq, k_flat, v_flat, Wo)


# device time: 39759 ns/iter; 1.6474x vs baseline; 1.6474x over previous
import jax
import jax.numpy as jnp
from jax import lax
from jax.experimental import pallas as pl
from jax.experimental.pallas import tpu as pltpu

N_DEV = 8
B = 2
SQ = 512
SKV = 512
HQ = 64
HQ_LOC = 8
DH = 64
D_MODEL = 768
HBLK = HQ_LOC * DH


def kernel(x, Wq, K_ext, V_ext, Wo):
    i = lax.axis_index("i")
    k_flat = K_ext.reshape(B, SKV, HQ * DH)
    v_flat = V_ext.reshape(B, SKV, HQ * DH)
    k_loc = lax.dynamic_slice_in_dim(k_flat, i * HBLK, HBLK, axis=2)
    k_loc = k_loc.transpose(0, 2, 1)
    v_loc = lax.dynamic_slice_in_dim(v_flat, i * HBLK, HBLK, axis=2)

    def body(x_ref, wq_ref, k_ref, v_ref, wo_ref, out_ref,
             acc_ref, rx_ref, ctx_ref, send_sems, recv_sems):
        my = lax.axis_index("i")
        CT = D_MODEL // 3

        barrier_sem = pltpu.get_barrier_semaphore()
        for mask in (1, 3, 4):
            pl.semaphore_signal(
                barrier_sem, inc=1,
                device_id=(jnp.bitwise_xor(my, mask),),
                device_id_type=pl.DeviceIdType.MESH)
        pl.semaphore_wait(barrier_sem, 3)

        x2 = x_ref[:].reshape(B * SQ, D_MODEL)
        q = jnp.dot(x2, wq_ref[:], preferred_element_type=jnp.float32)

        qi = lax.broadcasted_iota(jnp.int32, (SQ, SKV), 0)
        ki = lax.broadcasted_iota(jnp.int32, (SQ, SKV), 1)
        maskf = ((jnp.abs(qi - ki) <= 128) | (ki < 32) | (qi < 32)
                 ).astype(jnp.float32)

        for b in range(B):
            for h in range(HQ_LOC):
                q_bh = q[b * SQ:(b + 1) * SQ, h * DH:(h + 1) * DH]
                kt_bh = k_ref[b, h * DH:(h + 1) * DH, :]
                v_bh = v_ref[b, :, h * DH:(h + 1) * DH]
                s = jnp.dot(q_bh, kt_bh,
                            preferred_element_type=jnp.float32) * 0.125
                e = jnp.exp(s) * maskf
                ctx_ref[:, h * DH:(h + 1) * DH] = (
                    jnp.dot(e, v_bh, preferred_element_type=jnp.float32)
                    / jnp.sum(e, axis=1, keepdims=True))
            acc = jnp.dot(ctx_ref[:], wo_ref[:],
                          preferred_element_type=jnp.float32)
            for t in range(3):
                acc_ref[t, b * SQ:(b + 1) * SQ, :] = (
                    acc[:, CT * t:CT * (t + 1)].astype(jnp.bfloat16))

        d0 = jnp.bitwise_and(my, 1)
        d1 = jnp.bitwise_and(lax.shift_right_logical(my, 1), 1)
        d2 = jnp.bitwise_and(lax.shift_right_logical(my, 2), 1)
        coef = {1: jnp.bitwise_xor(d0, d1), 3: d1, 4: d2}
        perms = ((1, 3, 4), (3, 4, 1), (4, 1, 3))

        stage_params = []
        for t in range(3):
            m0, mA, mB = perms[t]
            al = coef[m0]
            keep = al * 512
            stage_params.append([
                (m0, (1 - al) * 512, 0),
                (mA, keep, 512),
                (mB, keep, 1024),
                (m0, keep, None),
            ])

        def make_rdma(t, s):
            mask, src_off, rx_off = stage_params[t][s]
            src = acc_ref.at[t, pl.ds(src_off, 512)]
            if rx_off is None:
                dst = acc_ref.at[t, pl.ds(src_off, 512)]
            else:
                dst = rx_ref.at[t, pl.ds(rx_off, 512)]
            return pltpu.make_async_remote_copy(
                src_ref=src, dst_ref=dst,
                send_sem=send_sems.at[t * 4 + s],
                recv_sem=recv_sems.at[t * 4 + s],
                device_id=(jnp.bitwise_xor(my, mask),),
                device_id_type=pl.DeviceIdType.MESH,
            )

        rd = [[None] * 4 for _ in range(3)]
        for t in range(3):
            rd[t][0] = make_rdma(t, 0)
            rd[t][0].start()
        for s in range(4):
            for t in range(3):
                rd[t][s].wait()
                if s < 3:
                    keep = stage_params[t][1][1]
                    rx_off = stage_params[t][s][2]
                    acc_ref[t, pl.ds(keep, 512), :] = (
                        acc_ref[t, pl.ds(keep, 512), :]
                        + rx_ref[t, pl.ds(rx_off, 512), :])
                if s < 3:
                    rd[t][s + 1] = make_rdma(t, s + 1)
                    rd[t][s + 1].start()

        for t in range(3):
            out_ref[:, :, CT * t:CT * (t + 1)] = (
                acc_ref[t].reshape(B, SQ, CT).astype(jnp.float32))

    return pl.pallas_call(
        body,
        out_shape=jax.ShapeDtypeStruct((B, SQ, D_MODEL), jnp.float32),
        in_specs=[pl.BlockSpec(memory_space=pltpu.VMEM)] * 5,
        out_specs=pl.BlockSpec(memory_space=pltpu.VMEM),
        scratch_shapes=[
            pltpu.VMEM((3, B * SQ, D_MODEL // 3), jnp.bfloat16),
            pltpu.VMEM((3, 1536, D_MODEL // 3), jnp.bfloat16),
            pltpu.VMEM((SQ, HBLK), jnp.float32),
            pltpu.SemaphoreType.DMA((12,)),
            pltpu.SemaphoreType.DMA((12,)),
        ],
        compiler_params=pltpu.CompilerParams(collective_id=0),
    )(x, Wq, k_loc, v_loc, Wo)
